# baseline (speedup 1.0000x reference)
import functools
import math

import numpy as np
import jax
import jax.numpy as jnp
from jax import lax
from jax.experimental import pallas as pl
from jax.experimental.pallas import tpu as pltpu
from jax.experimental.pallas import tpu_sc as plsc

VOCAB = 100000
WORD_VEC = 300
WPAD = 384
HIDDEN = 768
MAX_POS = 512
EPS = 1e-12

B, L = 1024, 50
NTOK = B * L


def _make_pe(n_filters=HIDDEN, max_len=MAX_POS * 2):
    position = np.arange(0, max_len).astype(np.float32)[:, None]
    div_term = np.exp(
        np.arange(0, n_filters, 2).astype(np.float32) * -(math.log(10000.0) / n_filters)
    )
    pe = np.zeros((max_len, n_filters), dtype=np.float32)
    pe[:, 0::2] = np.sin(position * div_term)
    pe[:, 1::2] = np.cos(position * div_term)
    return pe


NC, NS = 2, 16
NW = NC * NS
PER_W = NTOK // NW
CHUNK = 80
NCH = PER_W // CHUNK


@functools.lru_cache(maxsize=None)
def _build_sc_gather():
    mesh = plsc.VectorSubcoreMesh(core_axis_name="c", subcore_axis_name="s")

    @functools.partial(
        pl.kernel,
        mesh=mesh,
        out_type=jax.ShapeDtypeStruct((NTOK, WPAD), jnp.float32),
        scratch_types=[
            pltpu.VMEM((NCH, CHUNK), jnp.int32),
            pltpu.VMEM((CHUNK, WPAD), jnp.float32),
            pltpu.SemaphoreType.DMA,
        ],
    )
    def _sc_gather(table_hbm, idx_hbm, out_hbm, idx_v, rows_v, sem):
        wid = lax.axis_index("s") * NC + lax.axis_index("c")
        base = wid * PER_W
        pltpu.sync_copy(idx_hbm.at[wid], idx_v)

        def body(j, carry):
            pltpu.async_copy(table_hbm.at[idx_v.at[j]], rows_v, sem).wait()
            row0 = pl.multiple_of(base + j * CHUNK, CHUNK)
            pltpu.sync_copy(rows_v, out_hbm.at[pl.ds(row0, CHUNK)])
            return carry

        lax.fori_loop(0, NCH, body, 0)

    return _sc_gather


RB = 2000
PAD_GRID = VOCAB // RB


def _pad_body(t_ref, o_ref):
    x = t_ref[...]
    o_ref[...] = jnp.concatenate(
        [x, jnp.zeros((RB, WPAD - WORD_VEC), jnp.float32)], axis=1
    )


def _tc_pad(word_table):
    return pl.pallas_call(
        _pad_body,
        grid=(PAD_GRID,),
        in_specs=[pl.BlockSpec((RB, WORD_VEC), lambda i: (i, 0))],
        out_specs=pl.BlockSpec((RB, WPAD), lambda i: (i, 0)),
        out_shape=jax.ShapeDtypeStruct((VOCAB, WPAD), jnp.float32),
        compiler_params=pltpu.CompilerParams(
            dimension_semantics=("parallel",),
        ),
    )(word_table)


BBB = 16
GRID = B // BBB
TB = L * BBB
INV_WV = 1.0 / WORD_VEC


def _ln(x, g, b):
    mu = jnp.mean(x, axis=-1, keepdims=True)
    xc = x - mu
    var = jnp.mean(xc * xc, axis=-1, keepdims=True)
    return xc * lax.rsqrt(var + EPS) * g + b


def _tc_body(we_ref, vf_ref, tm_ref, W2_ref, r_ref, c_ref, g2_ref,
             b2_ref, vg_ref, vb_ref, fg_ref, fb_ref, tt_ref, pe_ref, o_ref):
    x = we_ref[...].reshape(TB, WPAD)
    s1 = jnp.sum(x, axis=-1, keepdims=True)
    s2 = jnp.sum(x * x, axis=-1, keepdims=True)
    mu = s1 * INV_WV
    var = s2 * INV_WV - mu * mu
    rs = lax.rsqrt(var + EPS)
    xw = jnp.dot(x.astype(jnp.bfloat16), W2_ref[...],
                 preferred_element_type=jnp.float32)
    h = rs * (xw - mu * r_ref[...]) + c_ref[...]
    h = _ln(jnp.maximum(h, 0.0), g2_ref[...], b2_ref[...])
    h3 = h.reshape(L, BBB, HIDDEN)
    ve = _ln(vf_ref[...], vg_ref[...].reshape(1, 1, HIDDEN),
             vb_ref[...].reshape(1, 1, HIDDEN))
    tt0 = tt_ref[0:1, :].reshape(1, 1, HIDDEN)
    tt1 = tt_ref[1:2, :].reshape(1, 1, HIDDEN)
    tt2 = 2.0 * (tt0 + tm_ref[...] * (tt1 - tt0))
    emb = h3 + tt2 + ve + pe_ref[...]
    o_ref[...] = _ln(emb, fg_ref[...].reshape(1, 1, HIDDEN),
                     fb_ref[...].reshape(1, 1, HIDDEN))


_PE_BLOCK = _make_pe()[:L][:, None, :]


def _tc_fused(we3, vf, tm, W2, r, c, g2, b2, vg, vb, fg, fb, tt):
    col3 = lambda i: (0, i, 0)
    fixed = lambda i: (0, 0)
    fixed3 = lambda i: (0, 0, 0)
    return pl.pallas_call(
        _tc_body,
        grid=(GRID,),
        in_specs=[
            pl.BlockSpec((L, BBB, WPAD), col3),
            pl.BlockSpec((L, BBB, HIDDEN), col3),
            pl.BlockSpec((L, BBB, 1), col3),
            pl.BlockSpec((WPAD, HIDDEN), fixed),
            pl.BlockSpec((1, HIDDEN), fixed),
            pl.BlockSpec((1, HIDDEN), fixed),
            pl.BlockSpec((1, HIDDEN), fixed),
            pl.BlockSpec((1, HIDDEN), fixed),
            pl.BlockSpec((1, HIDDEN), fixed),
            pl.BlockSpec((1, HIDDEN), fixed),
            pl.BlockSpec((1, HIDDEN), fixed),
            pl.BlockSpec((1, HIDDEN), fixed),
            pl.BlockSpec((2, HIDDEN), fixed),
            pl.BlockSpec((L, 1, HIDDEN), fixed3),
        ],
        out_specs=pl.BlockSpec((L, BBB, HIDDEN), col3),
        out_shape=jax.ShapeDtypeStruct((L, B, HIDDEN), jnp.float32),
        compiler_params=pltpu.CompilerParams(
            dimension_semantics=("parallel",),
        ),
    )(we3, vf, tm, W2, r, c, g2, b2, vg, vb, fg, fb, tt,
      jnp.asarray(_PE_BLOCK))


def kernel(input_ids, video_features, token_type_ids, word_table, tt_table, W,
           b, ln1_g, ln1_b, ln2_g, ln2_b, vln_g, vln_b, lnf_g, lnf_b):
    idx = input_ids.T.reshape(NW, NCH, CHUNK).astype(jnp.int32)
    wt = jnp.pad(word_table, ((0, 0), (0, WPAD - WORD_VEC)))
    we3 = _build_sc_gather()(wt, idx).reshape(L, B, WPAD)

    W2 = jnp.pad(ln1_g[:, None] * W, ((0, WPAD - WORD_VEC), (0, 0))).astype(
        jnp.bfloat16)
    r = (ln1_g @ W).reshape(1, HIDDEN)
    c = (ln1_b @ W + b).reshape(1, HIDDEN)

    tm = token_type_ids.T.astype(jnp.float32)[..., None]
    vf = jnp.transpose(video_features, (1, 0, 2))
    out3 = _tc_fused(
        we3, vf, tm, W2, r, c,
        ln2_g.reshape(1, HIDDEN), ln2_b.reshape(1, HIDDEN),
        vln_g.reshape(1, HIDDEN), vln_b.reshape(1, HIDDEN),
        lnf_g.reshape(1, HIDDEN), lnf_b.reshape(1, HIDDEN),
        tt_table,
    )
    return jnp.transpose(out3, (1, 0, 2))


# speedup vs baseline: 1.6703x; 1.6703x over previous
import functools
import math

import numpy as np
import jax
import jax.numpy as jnp
from jax import lax
from jax.experimental import pallas as pl
from jax.experimental.pallas import tpu as pltpu
from jax.experimental.pallas import tpu_sc as plsc

VOCAB = 100000
WORD_VEC = 300
WPAD = 384
HIDDEN = 768
MAX_POS = 512
EPS = 1e-12

B, L = 1024, 50
NTOK = B * L


def _make_pe(n_filters=HIDDEN, max_len=MAX_POS * 2):
    position = np.arange(0, max_len).astype(np.float32)[:, None]
    div_term = np.exp(
        np.arange(0, n_filters, 2).astype(np.float32) * -(math.log(10000.0) / n_filters)
    )
    pe = np.zeros((max_len, n_filters), dtype=np.float32)
    pe[:, 0::2] = np.sin(position * div_term)
    pe[:, 1::2] = np.cos(position * div_term)
    return pe


NC, NS = 2, 16
NW = NC * NS
PER_W = NTOK // NW
CHUNK = 80
NCH = PER_W // CHUNK


@functools.lru_cache(maxsize=None)
def _build_sc_gather():
    mesh = plsc.VectorSubcoreMesh(core_axis_name="c", subcore_axis_name="s")

    @functools.partial(
        pl.kernel,
        mesh=mesh,
        out_type=jax.ShapeDtypeStruct((NTOK, WPAD), jnp.float32),
        scratch_types=[
            pltpu.VMEM((NCH, CHUNK), jnp.int32),
            pltpu.VMEM((CHUNK, WPAD), jnp.float32),
            pltpu.SemaphoreType.DMA,
        ],
    )
    def _sc_gather(table_hbm, idx_hbm, out_hbm, idx_v, rows_v, sem):
        wid = lax.axis_index("s") * NC + lax.axis_index("c")
        base = wid * PER_W
        pltpu.sync_copy(idx_hbm.at[wid], idx_v)

        def body(j, carry):
            pltpu.async_copy(table_hbm.at[idx_v.at[j]], rows_v, sem).wait()
            row0 = pl.multiple_of(base + j * CHUNK, CHUNK)
            pltpu.sync_copy(rows_v, out_hbm.at[pl.ds(row0, CHUNK)])
            return carry

        lax.fori_loop(0, NCH, body, 0)

    return _sc_gather


RB = 2000
PAD_GRID = VOCAB // RB


def _pad_body(t_ref, o_ref):
    x = t_ref[...]
    o_ref[...] = jnp.concatenate(
        [x, jnp.zeros((RB, WPAD - WORD_VEC), jnp.float32)], axis=1
    )


def _tc_pad(word_table):
    return pl.pallas_call(
        _pad_body,
        grid=(PAD_GRID,),
        in_specs=[pl.BlockSpec((RB, WORD_VEC), lambda i: (i, 0))],
        out_specs=pl.BlockSpec((RB, WPAD), lambda i: (i, 0)),
        out_shape=jax.ShapeDtypeStruct((VOCAB, WPAD), jnp.float32),
        compiler_params=pltpu.CompilerParams(
            dimension_semantics=("parallel",),
        ),
    )(word_table)


BBB = 16
GRID = B // BBB
TB = L * BBB
INV_WV = 1.0 / WORD_VEC


def _ln(x, g, b):
    mu = jnp.mean(x, axis=-1, keepdims=True)
    xc = x - mu
    var = jnp.mean(xc * xc, axis=-1, keepdims=True)
    return xc * lax.rsqrt(var + EPS) * g + b


def _tc_body(we_ref, vf_ref, tm_ref, W2_ref, r_ref, c_ref, g2_ref,
             b2_ref, vg_ref, vb_ref, fg_ref, fb_ref, tt_ref, pe_ref, o_ref):
    x = we_ref[...].reshape(TB, WPAD)
    s1 = jnp.sum(x, axis=-1, keepdims=True)
    s2 = jnp.sum(x * x, axis=-1, keepdims=True)
    mu = s1 * INV_WV
    var = s2 * INV_WV - mu * mu
    rs = lax.rsqrt(var + EPS)
    xw = jnp.dot(x.astype(jnp.bfloat16), W2_ref[...],
                 preferred_element_type=jnp.float32)
    h = rs * (xw - mu * r_ref[...]) + c_ref[...]
    h = _ln(jnp.maximum(h, 0.0), g2_ref[...], b2_ref[...])
    h3 = h.reshape(L, BBB, HIDDEN)
    ve = _ln(vf_ref[...], vg_ref[...].reshape(1, 1, HIDDEN),
             vb_ref[...].reshape(1, 1, HIDDEN))
    tt0 = tt_ref[0:1, :].reshape(1, 1, HIDDEN)
    tt1 = tt_ref[1:2, :].reshape(1, 1, HIDDEN)
    tt2 = 2.0 * (tt0 + tm_ref[...] * (tt1 - tt0))
    emb = h3 + tt2 + ve + pe_ref[...]
    o_ref[...] = _ln(emb, fg_ref[...].reshape(1, 1, HIDDEN),
                     fb_ref[...].reshape(1, 1, HIDDEN))


_PE_BLOCK = _make_pe()[:L][:, None, :]


def _tc_fused(we3, vf, tm, W2, r, c, g2, b2, vg, vb, fg, fb, tt):
    col3 = lambda i: (0, i, 0)
    fixed = lambda i: (0, 0)
    fixed3 = lambda i: (0, 0, 0)
    return pl.pallas_call(
        _tc_body,
        grid=(GRID,),
        in_specs=[
            pl.BlockSpec((L, BBB, WPAD), col3),
            pl.BlockSpec((L, BBB, HIDDEN), col3),
            pl.BlockSpec((L, BBB, 1), col3),
            pl.BlockSpec((WPAD, HIDDEN), fixed),
            pl.BlockSpec((1, HIDDEN), fixed),
            pl.BlockSpec((1, HIDDEN), fixed),
            pl.BlockSpec((1, HIDDEN), fixed),
            pl.BlockSpec((1, HIDDEN), fixed),
            pl.BlockSpec((1, HIDDEN), fixed),
            pl.BlockSpec((1, HIDDEN), fixed),
            pl.BlockSpec((1, HIDDEN), fixed),
            pl.BlockSpec((1, HIDDEN), fixed),
            pl.BlockSpec((2, HIDDEN), fixed),
            pl.BlockSpec((L, 1, HIDDEN), fixed3),
        ],
        out_specs=pl.BlockSpec((L, BBB, HIDDEN), col3),
        out_shape=jax.ShapeDtypeStruct((L, B, HIDDEN), jnp.float32),
        compiler_params=pltpu.CompilerParams(
            dimension_semantics=("parallel",),
        ),
    )(we3, vf, tm, W2, r, c, g2, b2, vg, vb, fg, fb, tt,
      jnp.asarray(_PE_BLOCK))


def kernel(input_ids, video_features, token_type_ids, word_table, tt_table, W,
           b, ln1_g, ln1_b, ln2_g, ln2_b, vln_g, vln_b, lnf_g, lnf_b):
    idx = input_ids.T.reshape(NW, NCH, CHUNK).astype(jnp.int32)
    wt = _tc_pad(word_table)
    we3 = _build_sc_gather()(wt, idx).reshape(L, B, WPAD)

    W2 = jnp.pad(ln1_g[:, None] * W, ((0, WPAD - WORD_VEC), (0, 0))).astype(
        jnp.bfloat16)
    r = (ln1_g @ W).reshape(1, HIDDEN)
    c = (ln1_b @ W + b).reshape(1, HIDDEN)

    tm = token_type_ids.T.astype(jnp.float32)[..., None]
    vf = jnp.transpose(video_features, (1, 0, 2))
    out3 = _tc_fused(
        we3, vf, tm, W2, r, c,
        ln2_g.reshape(1, HIDDEN), ln2_b.reshape(1, HIDDEN),
        vln_g.reshape(1, HIDDEN), vln_b.reshape(1, HIDDEN),
        lnf_g.reshape(1, HIDDEN), lnf_b.reshape(1, HIDDEN),
        tt_table,
    )
    return jnp.transpose(out3, (1, 0, 2))
